# baseline (speedup 1.0000x reference)
import functools

import jax
import jax.numpy as jnp
from jax.experimental import pallas as pl
from jax.experimental.pallas import tpu as pltpu

TOKENS = 8192
INPUT_DIM = 2048
L = 4
HIDDEN_DIM = 4096
NUM_EXPERTS = 64
TOPK = 8

TM_A = 256
TM = 1024
TN = 512


def _pool_body(x_ref, o_ref):
    xb = x_ref[...].reshape(TM_A * 64, 128)
    s = xb + pltpu.roll(xb, 126, 1)
    s = s + pltpu.roll(s, 127, 1)
    rows = jax.lax.broadcasted_iota(jnp.int32, (128, 32), 0)
    cols = jax.lax.broadcasted_iota(jnp.int32, (128, 32), 1)
    sel = jnp.where(rows == cols * L, 1.0 / L, 0.0).astype(jnp.bfloat16)
    out = jnp.dot(s.astype(jnp.bfloat16), sel, preferred_element_type=jnp.float32)
    o_ref[...] = out.astype(jnp.bfloat16).reshape(TM_A, 64, 32)


def _mlp_body(xp_ref, w1_ref, b1_ref, w2_ref, b2_ref, ow_ref, oi_ref, h_ref):
    n = pl.program_id(1)
    h = jnp.dot(xp_ref[...], w1_ref[...], preferred_element_type=jnp.float32)
    h_ref[:, pl.ds(n * TN, TN)] = jnp.maximum(h + b1_ref[...], 0.0).astype(jnp.bfloat16)

    @pl.when(n == HIDDEN_DIM // TN - 1)
    def _():
        logits = jnp.dot(h_ref[...], w2_ref[...],
                         preferred_element_type=jnp.float32) + b2_ref[...]
        lane = jax.lax.broadcasted_iota(jnp.int32, (TM, NUM_EXPERTS), 1)
        work = logits
        vals, inds = [], []
        for _ in range(TOPK):
            m = jnp.max(work, axis=1, keepdims=True)
            cand = jnp.where(work == m, lane, NUM_EXPERTS)
            am = jnp.min(cand, axis=1, keepdims=True)
            vals.append(m)
            inds.append(am)
            work = jnp.where(lane == am, -jnp.inf, work)
        w = jnp.concatenate(vals, axis=1)
        e = jnp.exp(w - w[:, 0:1])
        ow_ref[...] = e / jnp.sum(e, axis=1, keepdims=True)
        oi_ref[...] = jnp.concatenate(inds, axis=1).astype(jnp.int32)


@functools.partial(jax.jit, static_argnames=("interpret",))
def kernel(x, W1, b1, W2, b2, interpret=False):
    x3 = x.reshape(TOKENS, 64, 128)
    xp = pl.pallas_call(
        _pool_body,
        grid=(TOKENS // TM_A,),
        in_specs=[pl.BlockSpec((TM_A, 64, 128), lambda m: (m, 0, 0))],
        out_specs=pl.BlockSpec((TM_A, 64, 32), lambda m: (m, 0, 0)),
        out_shape=jax.ShapeDtypeStruct((TOKENS, 64, 32), jnp.bfloat16),
        interpret=interpret,
    )(x3)
    xp = xp.reshape(TOKENS, INPUT_DIM)

    w1b = W1.astype(jnp.bfloat16)
    w2b = W2.astype(jnp.bfloat16)
    b1r = b1.reshape(1, HIDDEN_DIM)
    b2r = b2.reshape(1, NUM_EXPERTS)

    grid = (TOKENS // TM, HIDDEN_DIM // TN)
    weights, indices = pl.pallas_call(
        _mlp_body,
        grid=grid,
        in_specs=[
            pl.BlockSpec((TM, INPUT_DIM), lambda m, n: (m, 0)),
            pl.BlockSpec((INPUT_DIM, TN), lambda m, n: (0, n)),
            pl.BlockSpec((1, TN), lambda m, n: (0, n)),
            pl.BlockSpec((HIDDEN_DIM, NUM_EXPERTS), lambda m, n: (0, 0)),
            pl.BlockSpec((1, NUM_EXPERTS), lambda m, n: (0, 0)),
        ],
        out_specs=(
            pl.BlockSpec((TM, TOPK), lambda m, n: (m, 0)),
            pl.BlockSpec((TM, TOPK), lambda m, n: (m, 0)),
        ),
        out_shape=(
            jax.ShapeDtypeStruct((TOKENS, TOPK), jnp.float32),
            jax.ShapeDtypeStruct((TOKENS, TOPK), jnp.int32),
        ),
        scratch_shapes=[pltpu.VMEM((TM, HIDDEN_DIM), jnp.bfloat16)],
        compiler_params=pltpu.CompilerParams(
            dimension_semantics=("arbitrary", "arbitrary"),
        ),
        interpret=interpret,
    )(xp, w1b, b1r, w2b, b2r)
    return weights, indices


# speedup vs baseline: 3.4626x; 3.4626x over previous
import functools

import jax
import jax.numpy as jnp
from jax.experimental import pallas as pl
from jax.experimental.pallas import tpu as pltpu

TOKENS = 8192
INPUT_DIM = 2048
L = 4
HIDDEN_DIM = 4096
NUM_EXPERTS = 64
TOPK = 8

TM_A = 256
TM = 1024
TN = 512


def _pool_body(x_ref, o_ref):
    xb = x_ref[...]
    s = (xb[:, 0, :] + xb[:, 2, :]) + (xb[:, 1, :] + xb[:, 3, :])
    o_ref[...] = (s * (1.0 / L)).astype(jnp.bfloat16)


def _mlp_body(xp_ref, w1_ref, b1_ref, w2_ref, b2_ref, ow_ref, oi_ref, h_ref):
    n = pl.program_id(1)
    h = jnp.dot(xp_ref[...], w1_ref[...], preferred_element_type=jnp.float32)
    h_ref[:, pl.ds(n * TN, TN)] = jnp.maximum(h + b1_ref[...], 0.0).astype(jnp.bfloat16)

    @pl.when(n == HIDDEN_DIM // TN - 1)
    def _():
        logits = jnp.dot(h_ref[...], w2_ref[...],
                         preferred_element_type=jnp.float32) + b2_ref[...]
        lane = jax.lax.broadcasted_iota(jnp.int32, (TM, NUM_EXPERTS), 1)
        work = logits
        vals, inds = [], []
        for _ in range(TOPK):
            m = jnp.max(work, axis=1, keepdims=True)
            cand = jnp.where(work == m, lane, NUM_EXPERTS)
            am = jnp.min(cand, axis=1, keepdims=True)
            vals.append(m)
            inds.append(am)
            work = jnp.where(lane == am, -jnp.inf, work)
        w = jnp.concatenate(vals, axis=1)
        e = jnp.exp(w - w[:, 0:1])
        ow_ref[...] = e / jnp.sum(e, axis=1, keepdims=True)
        oi_ref[...] = jnp.concatenate(inds, axis=1).astype(jnp.int32)


@functools.partial(jax.jit, static_argnames=("interpret",))
def kernel(x, W1, b1, W2, b2, interpret=False):
    xt = jnp.transpose(x, (0, 2, 1))
    xp = pl.pallas_call(
        _pool_body,
        grid=(TOKENS // TM_A,),
        in_specs=[pl.BlockSpec((TM_A, L, INPUT_DIM), lambda m: (m, 0, 0))],
        out_specs=pl.BlockSpec((TM_A, INPUT_DIM), lambda m: (m, 0)),
        out_shape=jax.ShapeDtypeStruct((TOKENS, INPUT_DIM), jnp.bfloat16),
        interpret=interpret,
    )(xt)

    w1b = W1.astype(jnp.bfloat16)
    w2b = W2.astype(jnp.bfloat16)
    b1r = b1.reshape(1, HIDDEN_DIM)
    b2r = b2.reshape(1, NUM_EXPERTS)

    grid = (TOKENS // TM, HIDDEN_DIM // TN)
    weights, indices = pl.pallas_call(
        _mlp_body,
        grid=grid,
        in_specs=[
            pl.BlockSpec((TM, INPUT_DIM), lambda m, n: (m, 0)),
            pl.BlockSpec((INPUT_DIM, TN), lambda m, n: (0, n)),
            pl.BlockSpec((1, TN), lambda m, n: (0, n)),
            pl.BlockSpec((HIDDEN_DIM, NUM_EXPERTS), lambda m, n: (0, 0)),
            pl.BlockSpec((1, NUM_EXPERTS), lambda m, n: (0, 0)),
        ],
        out_specs=(
            pl.BlockSpec((TM, TOPK), lambda m, n: (m, 0)),
            pl.BlockSpec((TM, TOPK), lambda m, n: (m, 0)),
        ),
        out_shape=(
            jax.ShapeDtypeStruct((TOKENS, TOPK), jnp.float32),
            jax.ShapeDtypeStruct((TOKENS, TOPK), jnp.int32),
        ),
        scratch_shapes=[pltpu.VMEM((TM, HIDDEN_DIM), jnp.bfloat16)],
        compiler_params=pltpu.CompilerParams(
            dimension_semantics=("arbitrary", "arbitrary"),
        ),
        interpret=interpret,
    )(xp, w1b, b1r, w2b, b2r)
    return weights, indices


# speedup vs baseline: 3.6567x; 1.0561x over previous
import functools

import jax
import jax.numpy as jnp
from jax.experimental import pallas as pl
from jax.experimental.pallas import tpu as pltpu

TOKENS = 8192
INPUT_DIM = 2048
L = 4
HIDDEN_DIM = 4096
NUM_EXPERTS = 64
TOPK = 8

TM_A = 256
TM = 2048
TN = 512


def _pool_body(x_ref, o_ref):
    xb = x_ref[...]
    s = (xb[:, 0, :] + xb[:, 2, :]) + (xb[:, 1, :] + xb[:, 3, :])
    o_ref[...] = (s * (1.0 / L)).astype(jnp.bfloat16)


def _mlp_body(xp_ref, w1_ref, b1_ref, w2_ref, b2_ref, ow_ref, oi_ref, h_ref):
    n = pl.program_id(1)
    h = jnp.dot(xp_ref[...], w1_ref[...], preferred_element_type=jnp.float32)
    h_ref[:, pl.ds(n * TN, TN)] = jnp.maximum(h + b1_ref[...], 0.0).astype(jnp.bfloat16)

    @pl.when(n == HIDDEN_DIM // TN - 1)
    def _():
        logits = jnp.dot(h_ref[...], w2_ref[...],
                         preferred_element_type=jnp.float32) + b2_ref[...]
        lane = jax.lax.broadcasted_iota(jnp.int32, (TM, NUM_EXPERTS), 1)
        work = logits
        vals, inds = [], []
        for _ in range(TOPK):
            m = jnp.max(work, axis=1, keepdims=True)
            cand = jnp.where(work == m, lane, NUM_EXPERTS)
            am = jnp.min(cand, axis=1, keepdims=True)
            vals.append(m)
            inds.append(am)
            work = jnp.where(lane == am, -jnp.inf, work)
        w = jnp.concatenate(vals, axis=1)
        e = jnp.exp(w - w[:, 0:1])
        ow_ref[...] = e / jnp.sum(e, axis=1, keepdims=True)
        oi_ref[...] = jnp.concatenate(inds, axis=1).astype(jnp.int32)


@functools.partial(jax.jit, static_argnames=("interpret",))
def kernel(x, W1, b1, W2, b2, interpret=False):
    xt = jnp.transpose(x, (0, 2, 1))
    xp = pl.pallas_call(
        _pool_body,
        grid=(TOKENS // TM_A,),
        in_specs=[pl.BlockSpec((TM_A, L, INPUT_DIM), lambda m: (m, 0, 0))],
        out_specs=pl.BlockSpec((TM_A, INPUT_DIM), lambda m: (m, 0)),
        out_shape=jax.ShapeDtypeStruct((TOKENS, INPUT_DIM), jnp.bfloat16),
        interpret=interpret,
    )(xt)

    w1b = W1.astype(jnp.bfloat16)
    w2b = W2.astype(jnp.bfloat16)
    b1r = b1.reshape(1, HIDDEN_DIM)
    b2r = b2.reshape(1, NUM_EXPERTS)

    grid = (TOKENS // TM, HIDDEN_DIM // TN)
    weights, indices = pl.pallas_call(
        _mlp_body,
        grid=grid,
        in_specs=[
            pl.BlockSpec((TM, INPUT_DIM), lambda m, n: (m, 0)),
            pl.BlockSpec((INPUT_DIM, TN), lambda m, n: (0, n)),
            pl.BlockSpec((1, TN), lambda m, n: (0, n)),
            pl.BlockSpec((HIDDEN_DIM, NUM_EXPERTS), lambda m, n: (0, 0)),
            pl.BlockSpec((1, NUM_EXPERTS), lambda m, n: (0, 0)),
        ],
        out_specs=(
            pl.BlockSpec((TM, TOPK), lambda m, n: (m, 0)),
            pl.BlockSpec((TM, TOPK), lambda m, n: (m, 0)),
        ),
        out_shape=(
            jax.ShapeDtypeStruct((TOKENS, TOPK), jnp.float32),
            jax.ShapeDtypeStruct((TOKENS, TOPK), jnp.int32),
        ),
        scratch_shapes=[pltpu.VMEM((TM, HIDDEN_DIM), jnp.bfloat16)],
        compiler_params=pltpu.CompilerParams(
            dimension_semantics=("arbitrary", "arbitrary"),
        ),
        interpret=interpret,
    )(xp, w1b, b1r, w2b, b2r)
    return weights, indices
